# baseline (speedup 1.0000x reference)
import functools

import jax
import jax.numpy as jnp
import numpy as np
from jax import lax
from jax.experimental import pallas as pl
from jax.experimental.pallas import tpu as pltpu
from jax.experimental.pallas import tpu_sc as plsc

_FIELD_DIMS = [100000] * 26
_EMBED_DIM = 16
_BATCH = 16384
_NUM_FIELDS = len(_FIELD_DIMS)
_TOTAL = _BATCH * _NUM_FIELDS

_NC = 2
_NS = 16
_NW = _NC * _NS
_LANES = 16

_PER_W = _TOTAL // _NW
_CHUNK = 26 * 64
_NCHUNKS = _PER_W // _CHUNK


def _build_kernel():
    mesh = plsc.VectorSubcoreMesh(core_axis_name="c", subcore_axis_name="s")

    @functools.partial(
        pl.kernel,
        mesh=mesh,
        out_type=jax.ShapeDtypeStruct((_TOTAL, _EMBED_DIM), jnp.float32),
        scratch_types=[
            pltpu.VMEM((_CHUNK,), jnp.int32),
            pltpu.VMEM((_CHUNK,), jnp.int32),
            pltpu.VMEM((_CHUNK,), jnp.int32),
            pltpu.VMEM((_CHUNK, _EMBED_DIM), jnp.float32),
            pltpu.SemaphoreType.DMA,
        ],
        compiler_params=pltpu.CompilerParams(use_tc_tiling_on_sc=False),
    )
    def emb(x_hbm, off_hbm, w_hbm, out_hbm, xv, offv, idxv, rows, sem):
        wid = lax.axis_index("s") * _NC + lax.axis_index("c")
        base = wid * _PER_W
        pltpu.sync_copy(off_hbm, offv)

        def chunk_body(t, carry):
            g = base + t * _CHUNK
            pltpu.sync_copy(x_hbm.at[pl.ds(g, _CHUNK)], xv)

            def add_body(i, c):
                s = pl.ds(i * _LANES, _LANES)
                idxv[s] = xv[s] + offv[s]
                return c

            lax.fori_loop(0, _CHUNK // _LANES, add_body, 0, unroll=8)
            pltpu.async_copy(w_hbm.at[idxv], rows, sem).wait()
            pltpu.sync_copy(rows, out_hbm.at[pl.ds(g, _CHUNK)])
            return carry

        lax.fori_loop(0, _NCHUNKS, chunk_body, 0)

    return emb


_EMB = _build_kernel()


def kernel(x, W):
    offsets = np.concatenate(([0], np.cumsum(_FIELD_DIMS)[:-1])).astype(np.int32)
    off_tiled = jnp.asarray(np.tile(offsets, _CHUNK // _NUM_FIELDS))
    x_flat = x.astype(jnp.int32).reshape(_TOTAL)
    out = _EMB(x_flat, off_tiled, W)
    return out.reshape(_BATCH, _NUM_FIELDS, _EMBED_DIM)


# speedup vs baseline: 1.0003x; 1.0003x over previous
import functools

import jax
import jax.numpy as jnp
import numpy as np
from jax import lax
from jax.experimental import pallas as pl
from jax.experimental.pallas import tpu as pltpu
from jax.experimental.pallas import tpu_sc as plsc

_FIELD_DIMS = [100000] * 26
_EMBED_DIM = 16
_BATCH = 16384
_NUM_FIELDS = len(_FIELD_DIMS)
_TOTAL = _BATCH * _NUM_FIELDS

_NC = 2
_NS = 16
_NW = _NC * _NS
_LANES = 16

_TOTAL_ROWS_ = sum(_FIELD_DIMS)
_PER_W = _TOTAL // _NW
_CHUNK = 26 * 64
_NCHUNKS = _PER_W // _CHUNK


def _build_kernel():
    mesh = plsc.VectorSubcoreMesh(core_axis_name="c", subcore_axis_name="s")

    @functools.partial(
        pl.kernel,
        mesh=mesh,
        out_type=jax.ShapeDtypeStruct((_TOTAL, _EMBED_DIM), jnp.float32),
        scratch_types=[
            pltpu.VMEM((_CHUNK,), jnp.int32),
            pltpu.VMEM((_CHUNK,), jnp.int32),
            pltpu.VMEM((_CHUNK,), jnp.int32),
            pltpu.VMEM((_CHUNK, _EMBED_DIM), jnp.float32),
            pltpu.SemaphoreType.DMA,
        ],
        compiler_params=pltpu.CompilerParams(use_tc_tiling_on_sc=False),
    )
    def emb(x_hbm, off_hbm, w_hbm, out_hbm, xv, offv, idxv, rows, sem):
        wid = lax.axis_index("s") * _NC + lax.axis_index("c")
        base = wid * _PER_W
        pltpu.sync_copy(off_hbm, offv)

        def chunk_body(t, carry):
            g = base + t * _CHUNK
            pltpu.sync_copy(x_hbm.at[pl.ds(g, _CHUNK)], xv)

            def add_body(i, c):
                s = pl.ds(i * _LANES, _LANES)
                idxv[s] = xv[s] + offv[s]
                return c

            lax.fori_loop(0, _CHUNK // _LANES, add_body, 0, unroll=8)
            pltpu.async_copy(w_hbm.at[idxv], rows, sem).wait()
            pltpu.sync_copy(rows, out_hbm.at[pl.ds(g, _CHUNK)])
            return carry

        lax.fori_loop(0, _NCHUNKS, chunk_body, 0)

    return emb


_EMB = _build_kernel()


def kernel(x, W):
    offsets = np.concatenate(([0], np.cumsum(_FIELD_DIMS)[:-1])).astype(np.int32)
    off_tiled = jnp.asarray(np.tile(offsets, _CHUNK // _NUM_FIELDS))
    x_flat = x.astype(jnp.int32).reshape(_TOTAL)
    w128 = lax.optimization_barrier(W.reshape(_TOTAL_ROWS_ // 8, 8 * _EMBED_DIM))
    w_lin = w128.reshape(_TOTAL_ROWS_, _EMBED_DIM)
    out = _EMB(x_flat, off_tiled, w_lin)
    return out.reshape(_BATCH, _NUM_FIELDS, _EMBED_DIM)


# speedup vs baseline: 1.2103x; 1.2100x over previous
import functools

import jax
import jax.numpy as jnp
import numpy as np
from jax import lax
from jax.experimental import pallas as pl
from jax.experimental.pallas import tpu as pltpu
from jax.experimental.pallas import tpu_sc as plsc

_FIELD_DIMS = [100000] * 26
_EMBED_DIM = 16
_BATCH = 16384
_NUM_FIELDS = len(_FIELD_DIMS)
_TOTAL = _BATCH * _NUM_FIELDS
_TABLE_ROWS = sum(_FIELD_DIMS)

_NC = 2
_NS = 16
_NW = _NC * _NS
_LANES = 16

_B_PER_W = _BATCH // _NW
_PER_W = _TOTAL // _NW
_CHUNK = 26 * 64
_NCHUNKS = _PER_W // _CHUNK
_BBLK = 128


def _mesh():
    return plsc.VectorSubcoreMesh(core_axis_name="c", subcore_axis_name="s")


def _wid():
    return lax.axis_index("s") * _NC + lax.axis_index("c")


def _build_prep():
    @functools.partial(
        pl.kernel,
        mesh=_mesh(),
        out_type=jax.ShapeDtypeStruct((_TOTAL,), jnp.int32),
        scratch_types=[
            pltpu.VMEM((_NUM_FIELDS, _B_PER_W), jnp.int32),
            pltpu.VMEM((2 * _LANES,), jnp.int32),
            pltpu.VMEM((_PER_W,), jnp.int32),
        ],
        compiler_params=pltpu.CompilerParams(use_tc_tiling_on_sc=True, needs_layout_passes=False),
    )
    def prep(xt_hbm, off_hbm, idx_hbm, xv, offv, idxv):
        w = _wid()
        b0 = w * _B_PER_W
        pltpu.sync_copy(off_hbm, offv)
        pltpu.sync_copy(xt_hbm.at[:, pl.ds(b0, _B_PER_W)], xv)
        off_lo = offv[pl.ds(0, _LANES)]
        off_hi = offv[pl.ds(_LANES, _LANES)]
        f_lo = lax.iota(jnp.int32, _LANES)
        f_hi = f_lo + _LANES
        f_hi_c = jnp.minimum(f_hi, _NUM_FIELDS - 1)
        hi_mask = f_hi < _NUM_FIELDS

        def body(b, carry):
            bvec = jnp.full((_LANES,), b, jnp.int32)
            lo = plsc.load_gather(xv, [f_lo, bvec]) + off_lo
            hi = plsc.load_gather(xv, [f_hi_c, bvec]) + off_hi
            n0 = b * _NUM_FIELDS
            idxv[pl.ds(n0, _LANES)] = lo
            nvec = n0 + f_hi
            plsc.store_scatter(idxv, [nvec], hi, mask=hi_mask)
            return carry

        lax.fori_loop(0, _B_PER_W, body, 0, unroll=4)
        pltpu.sync_copy(idxv, idx_hbm.at[pl.ds(w * _PER_W, _PER_W)])

    return prep


def _build_gather():
    @functools.partial(
        pl.kernel,
        mesh=_mesh(),
        out_type=jax.ShapeDtypeStruct((_TOTAL, _EMBED_DIM), jnp.float32),
        scratch_types=[
            pltpu.VMEM((_CHUNK,), jnp.int32),
            pltpu.VMEM((_CHUNK, _EMBED_DIM), jnp.float32),
            pltpu.SemaphoreType.DMA,
        ],
        compiler_params=pltpu.CompilerParams(use_tc_tiling_on_sc=False),
    )
    def gather(idx_hbm, w_hbm, out_hbm, idxv, rows, sem):
        base = _wid() * _PER_W

        def chunk_body(t, carry):
            g = base + t * _CHUNK
            pltpu.sync_copy(idx_hbm.at[pl.ds(g, _CHUNK)], idxv)
            pltpu.async_copy(w_hbm.at[idxv], rows, sem).wait()
            pltpu.sync_copy(rows, out_hbm.at[pl.ds(g, _CHUNK)])
            return carry

        lax.fori_loop(0, _NCHUNKS, chunk_body, 0)

    return gather


def _build_format():
    words_per_blk = _BBLK * _NUM_FIELDS * _EMBED_DIM

    @functools.partial(
        pl.kernel,
        mesh=_mesh(),
        out_type=jax.ShapeDtypeStruct((_NUM_FIELDS, _EMBED_DIM, _BATCH), jnp.float32),
        scratch_types=[
            pltpu.VMEM((words_per_blk,), jnp.float32),
            pltpu.VMEM((_NUM_FIELDS, _EMBED_DIM, _BBLK), jnp.float32),
        ],
        compiler_params=pltpu.CompilerParams(use_tc_tiling_on_sc=True, needs_layout_passes=False),
    )
    def fmt(lin_hbm, out_hbm, linv, slab):
        w = _wid()

        def blk_body(t, carry):
            b0 = w * _B_PER_W + t * _BBLK
            pltpu.sync_copy(lin_hbm.at[pl.ds(b0 * _NUM_FIELDS * _EMBED_DIM, words_per_blk)], linv)

            def f_body(f, c2):
                for j in range(_EMBED_DIM):
                    for bb in range(_BBLK // _LANES):
                        bvec = lax.iota(jnp.int32, _LANES) + bb * _LANES
                        src = (bvec * _NUM_FIELDS + f) * _EMBED_DIM + j
                        slab[f, j, pl.ds(bb * _LANES, _LANES)] = plsc.load_gather(linv, [src])
                return c2

            lax.fori_loop(0, _NUM_FIELDS, f_body, 0)
            pltpu.sync_copy(slab, out_hbm.at[:, :, pl.ds(b0, _BBLK)])
            return carry

        lax.fori_loop(0, _B_PER_W // _BBLK, blk_body, 0)

    return fmt


_PREP = _build_prep()
_GATHER = _build_gather()
_FMT = _build_format()


def kernel(x, W):
    offsets = np.concatenate(([0], np.cumsum(_FIELD_DIMS)[:-1])).astype(np.int32)
    off_pad = np.zeros(2 * _LANES, np.int32)
    off_pad[:_NUM_FIELDS] = offsets
    xt = x.astype(jnp.int32).T
    idx = _PREP(xt, jnp.asarray(off_pad))
    w128 = lax.optimization_barrier(W.reshape(_TABLE_ROWS // 8, 8 * _EMBED_DIM))
    w_lin = w128.reshape(_TABLE_ROWS, _EMBED_DIM)
    out_lin = _GATHER(idx, w_lin)
    ot = _FMT(out_lin.reshape(_TOTAL * _EMBED_DIM))
    return ot.transpose(2, 0, 1)
